# Initial kernel scaffold; baseline (speedup 1.0000x reference)
import functools

import jax
import jax.numpy as jnp
from jax.experimental import pallas as pl
from jax.experimental.pallas import tpu as pltpu

N_TOKENS = 2048
D_MODEL = 1024
D_FF = 2048
N_EXPERTS = 8

TOK_CHUNK = 512


def _gate_kernel(x_ref, wg_ref, bg_ref, wtok_ref):
    x = x_ref[...]
    logits = jnp.dot(x, wg_ref[...], preferred_element_type=jnp.float32)
    logits = logits + bg_ref[...]
    iota_e = jax.lax.broadcasted_iota(jnp.int32, logits.shape, 1)
    m1 = jnp.max(logits, axis=1, keepdims=True)
    a1 = jnp.min(jnp.where(logits == m1, iota_e, N_EXPERTS), axis=1, keepdims=True)
    oh1 = iota_e == a1
    l2 = jnp.where(oh1, -jnp.inf, logits)
    m2 = jnp.max(l2, axis=1, keepdims=True)
    a2 = jnp.min(jnp.where(l2 == m2, iota_e, N_EXPERTS), axis=1, keepdims=True)
    oh2 = iota_e == a2
    w1 = 1.0 / (1.0 + jnp.exp(m2 - m1))
    w2 = 1.0 - w1
    wtok_ref[...] = jnp.where(oh1, w1, 0.0) + jnp.where(oh2, w2, 0.0)


def _expert_kernel(x_ref, wtok_ref, w1_ref, b1_ref, w2_ref, b2_ref, out_ref):
    e = pl.program_id(0)

    @pl.when(e == 0)
    def _init():
        out_ref[...] = jnp.zeros_like(out_ref)

    w1 = w1_ref[0]
    w2 = w2_ref[0]
    b1 = b1_ref[0]
    b2 = b2_ref[0]
    for c in range(N_TOKENS // TOK_CHUNK):
        sl = pl.ds(c * TOK_CHUNK, TOK_CHUNK)
        xs = x_ref[sl, :]
        h = jnp.dot(xs, w1, preferred_element_type=jnp.float32) + b1
        h = h * jax.nn.sigmoid(h)
        o = jnp.dot(h, w2, preferred_element_type=jnp.float32) + b2
        wt = wtok_ref[sl, :][:, e][:, None]
        out_ref[sl, :] += wt * o


def kernel(inputs, Wg, bg, W1, b1, W2, b2):
    wtok = pl.pallas_call(
        _gate_kernel,
        out_shape=jax.ShapeDtypeStruct((N_TOKENS, N_EXPERTS), jnp.float32),
    )(inputs, Wg, bg[None, :])

    out = pl.pallas_call(
        _expert_kernel,
        grid=(N_EXPERTS,),
        in_specs=[
            pl.BlockSpec((N_TOKENS, D_MODEL), lambda e: (0, 0)),
            pl.BlockSpec((N_TOKENS, N_EXPERTS), lambda e: (0, 0)),
            pl.BlockSpec((1, D_MODEL, D_FF), lambda e: (e, 0, 0)),
            pl.BlockSpec((1, D_FF), lambda e: (e, 0)),
            pl.BlockSpec((1, D_FF, D_MODEL), lambda e: (e, 0, 0)),
            pl.BlockSpec((1, D_MODEL), lambda e: (e, 0)),
        ],
        out_specs=pl.BlockSpec((N_TOKENS, D_MODEL), lambda e: (0, 0)),
        out_shape=jax.ShapeDtypeStruct((N_TOKENS, D_MODEL), jnp.float32),
    )(inputs, wtok, W1, b1, W2, b2)
    return out


# speedup vs baseline: 1.4015x; 1.4015x over previous
import functools

import jax
import jax.numpy as jnp
from jax.experimental import pallas as pl
from jax.experimental.pallas import tpu as pltpu

N_TOKENS = 2048
D_MODEL = 1024
D_FF = 2048
N_EXPERTS = 8

TOK_CHUNK = 512


def _gate_kernel(x_ref, wg_ref, bg_ref, wtok_ref):
    x = x_ref[...]
    logits = jnp.dot(x, wg_ref[...], preferred_element_type=jnp.float32)
    logits = logits + bg_ref[...]
    iota_e = jax.lax.broadcasted_iota(jnp.int32, logits.shape, 1)
    m1 = jnp.max(logits, axis=1, keepdims=True)
    a1 = jnp.min(jnp.where(logits == m1, iota_e, N_EXPERTS), axis=1, keepdims=True)
    oh1 = iota_e == a1
    l2 = jnp.where(oh1, -jnp.inf, logits)
    m2 = jnp.max(l2, axis=1, keepdims=True)
    a2 = jnp.min(jnp.where(l2 == m2, iota_e, N_EXPERTS), axis=1, keepdims=True)
    oh2 = iota_e == a2
    w1 = 1.0 / (1.0 + jnp.exp(m2 - m1))
    w2 = 1.0 - w1
    wtok_ref[...] = jnp.where(oh1, w1, 0.0) + jnp.where(oh2, w2, 0.0)


def _expert_kernel(x_ref, wtok_ref, w1_ref, b1_ref, w2_ref, b2_ref, out_ref):
    e = pl.program_id(0)
    f = pl.program_id(1)

    @pl.when(jnp.logical_and(e == 0, f == 0))
    def _init():
        out_ref[...] = jnp.zeros_like(out_ref)

    w1 = w1_ref[0]
    w2 = w2_ref[0]
    b1 = b1_ref[0, 0]
    b2 = b2_ref[0, 0]
    for c in range(N_TOKENS // TOK_CHUNK):
        sl = pl.ds(c * TOK_CHUNK, TOK_CHUNK)
        xs = x_ref[sl, :]
        h = jnp.dot(xs, w1, preferred_element_type=jnp.float32) + b1
        h = h * jax.nn.sigmoid(h)
        o = jnp.dot(h, w2, preferred_element_type=jnp.float32)
        wtok = wtok_ref[sl, :]
        lane = jax.lax.broadcasted_iota(jnp.int32, wtok.shape, 1)
        wt = jnp.sum(jnp.where(lane == e, wtok, 0.0), axis=1, keepdims=True)
        o = jnp.where(f == 0, o + b2, o)
        out_ref[sl, :] += wt * o


def kernel(inputs, Wg, bg, W1, b1, W2, b2):
    wtok = pl.pallas_call(
        _gate_kernel,
        out_shape=jax.ShapeDtypeStruct((N_TOKENS, N_EXPERTS), jnp.float32),
    )(inputs, Wg, bg[None, :])

    FF_BLK = D_FF // 2
    out = pl.pallas_call(
        _expert_kernel,
        grid=(N_EXPERTS, 2),
        in_specs=[
            pl.BlockSpec((N_TOKENS, D_MODEL), lambda e, f: (0, 0)),
            pl.BlockSpec((N_TOKENS, N_EXPERTS), lambda e, f: (0, 0)),
            pl.BlockSpec((1, D_MODEL, FF_BLK), lambda e, f: (e, 0, f)),
            pl.BlockSpec((1, 1, FF_BLK), lambda e, f: (e, 0, f)),
            pl.BlockSpec((1, FF_BLK, D_MODEL), lambda e, f: (e, f, 0)),
            pl.BlockSpec((1, 1, D_MODEL), lambda e, f: (e, 0, 0)),
        ],
        out_specs=pl.BlockSpec((N_TOKENS, D_MODEL), lambda e, f: (0, 0)),
        out_shape=jax.ShapeDtypeStruct((N_TOKENS, D_MODEL), jnp.float32),
    )(inputs, wtok, W1, b1[:, None, :], W2, b2[:, None, :])
    return out


# speedup vs baseline: 1.6271x; 1.1610x over previous
import jax
import jax.numpy as jnp
from jax import lax
from jax.experimental import pallas as pl
from jax.experimental.pallas import tpu as pltpu
from jax.experimental.pallas import tpu_sc as plsc

N_TOKENS = 2048
D_MODEL = 1024
D_FF = 2048
N_EXPERTS = 8

NC = 2
NS = 16
NW = NC * NS
L = 16

TPW = N_TOKENS // NW
B_ROWS = 256
LOG2_B = 8
NBLK = 24
PADTOT = NBLK * B_ROWS
META_LEN = 32


def _gate_kernel(x_ref, wg_ref, bg_ref, rout_ref, hist_ref):
    x = x_ref[...]
    logits = jnp.dot(x, wg_ref[...], preferred_element_type=jnp.float32)
    logits = logits + bg_ref[...]
    iota_e = lax.broadcasted_iota(jnp.int32, logits.shape, 1)
    m1 = jnp.max(logits, axis=1, keepdims=True)
    a1 = jnp.min(jnp.where(logits == m1, iota_e, N_EXPERTS), axis=1, keepdims=True)
    l2 = jnp.where(iota_e == a1, -jnp.inf, logits)
    m2 = jnp.max(l2, axis=1, keepdims=True)
    a2 = jnp.min(jnp.where(l2 == m2, iota_e, N_EXPERTS), axis=1, keepdims=True)
    w1 = 1.0 / (1.0 + jnp.exp(m2 - m1))
    w2 = 1.0 - w1
    rout_ref[...] = (
        jnp.where(iota_e == 0, a1.astype(jnp.float32), 0.0)
        + jnp.where(iota_e == 1, a2.astype(jnp.float32), 0.0)
        + jnp.where(iota_e == 2, w1, 0.0)
        + jnp.where(iota_e == 3, w2, 0.0)
    )
    iota16 = lax.broadcasted_iota(jnp.int32, (N_TOKENS, 16), 1)
    cnt16 = ((iota16 == a1).astype(jnp.int32)
             + (iota16 == a2).astype(jnp.int32))
    hist_ref[...] = jnp.sum(cnt16.reshape(NW, TPW, 16), axis=1)


def _wid():
    return lax.axis_index("s") * NC + lax.axis_index("c")


def _iota():
    return lax.iota(jnp.int32, 16)


def _sc_dispatch_body(rout_hbm, hist_hbm, x_hbm,
                      xg_hbm, meta_hbm, pos_hbm,
                      rout_v, histall_v,
                      idx0_v, idx1_v, xrows_v, meta_v, sem, sem2, semx):
    wid = _wid()
    lane = _iota()
    xload = pltpu.async_copy(x_hbm.at[pl.ds(wid * TPW, TPW)], xrows_v, semx)
    pltpu.sync_copy(rout_hbm.at[pl.ds(wid * TPW * N_EXPERTS, TPW * N_EXPERTS)],
                    rout_v)
    pltpu.sync_copy(hist_hbm, histall_v)

    tot = jnp.zeros((16,), jnp.int32)
    prior = jnp.zeros((16,), jnp.int32)
    for t in range(NW):
        row = histall_v[pl.ds(t * 16, 16)]
        tot = tot + row
        prior = prior + jnp.where(jnp.full((16,), t, jnp.int32) < wid, row, 0)
    padded = ((tot + (B_ROWS - 1)) >> LOG2_B) << LOG2_B
    offset = plsc.cumsum(padded) - padded
    base = offset + prior
    segend = offset + padded
    nvalid = jnp.sum(padded >> LOG2_B)

    @pl.when(wid == 0)
    def _meta():
        for c in range(META_LEN // 16):
            brow = (c * 16 + lane) * B_ROWS
            be = jnp.zeros((16,), jnp.int32)
            for e in range(N_EXPERTS):
                end_e = jnp.sum(jnp.where(lane == e, segend, 0))
                be = be + jnp.where(brow >= end_e, 1, 0)
            be = jnp.minimum(be, N_EXPERTS - 1)
            gpos = c * 16 + lane
            be = jnp.where(gpos == NBLK, jnp.full((16,), 1, jnp.int32) * nvalid, be)
            be = jnp.where(gpos > NBLK, 0, be)
            meta_v[pl.ds(c * 16, 16)] = be
        pltpu.sync_copy(meta_v, meta_hbm)

    basecur = base
    for c in range(TPW // 16):
        tokidx = c * 16 + lane
        for slot in range(2):
            ev = plsc.load_gather(rout_v, [tokidx * N_EXPERTS + slot]
                                  ).astype(jnp.int32)
            pos_c = jnp.zeros((16,), jnp.int32)
            for e in range(N_EXPERTS):
                m = ev == e
                r = plsc.cumsum(jnp.where(m, 1, 0)) - 1
                bs = jnp.sum(jnp.where(lane == e, basecur, 0))
                pos_c = jnp.where(m, bs + r, pos_c)
                cnt = plsc.all_reduce_population_count(m)
                basecur = basecur + jnp.where(lane == e, cnt, 0)
            if slot == 0:
                idx0_v[pl.ds(c * 16, 16)] = pos_c
            else:
                idx1_v[pl.ds(c * 16, 16)] = pos_c
    pltpu.sync_copy(idx0_v, pos_hbm.at[pl.ds(wid * TPW, TPW)])
    pltpu.sync_copy(idx1_v, pos_hbm.at[pl.ds(N_TOKENS + wid * TPW, TPW)])

    xload.wait()
    c0 = pltpu.async_copy(xrows_v, xg_hbm.at[idx0_v], sem)
    c1 = pltpu.async_copy(xrows_v, xg_hbm.at[idx1_v], sem2)
    c0.wait()
    c1.wait()


def _mm_kernel(meta_ref, xg_ref, w1_ref, b1_ref, w2_ref, b2_ref, buf_ref):
    b = pl.program_id(0)
    nvalid = meta_ref[NBLK]

    @pl.when(b < nvalid)
    def _compute():
        h = jnp.dot(xg_ref[...], w1_ref[0], preferred_element_type=jnp.float32)
        h = h + b1_ref[0, 0]
        h = h * jax.nn.sigmoid(h)
        o = jnp.dot(h, w2_ref[0], preferred_element_type=jnp.float32)
        buf_ref[...] = o + b2_ref[0, 0]


QB = 16
NQ = TPW // QB


def _sc_combine_body(buf_hbm, pos_hbm, rout_hbm, out_hbm,
                     rout_v, idx0_v, idx1_v,
                     g0a, g0b, g1a, g1b, ora, orb,
                     sg0a, sg0b, sg1a, sg1b, soa, sob):
    wid = _wid()
    pltpu.sync_copy(rout_hbm.at[pl.ds(wid * TPW * N_EXPERTS, TPW * N_EXPERTS)],
                    rout_v)
    pltpu.sync_copy(pos_hbm.at[pl.ds(wid * TPW, TPW)], idx0_v)
    pltpu.sync_copy(pos_hbm.at[pl.ds(N_TOKENS + wid * TPW, TPW)], idx1_v)
    g0 = [g0a, g0b]
    g1 = [g1a, g1b]
    orows = [ora, orb]
    sg0 = [sg0a, sg0b]
    sg1 = [sg1a, sg1b]
    so = [soa, sob]

    def start_gather(q):
        p = q % 2
        c0 = pltpu.async_copy(buf_hbm.at[idx0_v.at[pl.ds(q * QB, QB)]],
                              g0[p], sg0[p])
        c1 = pltpu.async_copy(buf_hbm.at[idx1_v.at[pl.ds(q * QB, QB)]],
                              g1[p], sg1[p])
        return c0, c1

    gath = [None] * NQ
    outw = [None] * NQ
    gath[0] = start_gather(0)
    gath[1] = start_gather(1)
    for q in range(NQ):
        p = q % 2
        gath[q][0].wait()
        gath[q][1].wait()
        if q >= 2:
            outw[q - 2].wait()

        def body(i, _, q=q, p=p):
            t_loc = q * QB + i
            w0 = plsc.load_gather(rout_v, [jnp.full((16,), t_loc * N_EXPERTS + 2,
                                                    jnp.int32)])
            w1 = plsc.load_gather(rout_v, [jnp.full((16,), t_loc * N_EXPERTS + 3,
                                                    jnp.int32)])
            for d in range(D_MODEL // 16):
                sl = pl.ds(d * 16, 16)
                orows[p][i, sl] = w0 * g0[p][i, sl] + w1 * g1[p][i, sl]
            return 0

        lax.fori_loop(0, QB, body, 0)
        outw[q] = pltpu.async_copy(
            orows[p], out_hbm.at[pl.ds(wid * TPW + q * QB, QB)], so[p])
        if q + 2 < NQ:
            gath[q + 2] = start_gather(q + 2)
    outw[NQ - 2].wait()
    outw[NQ - 1].wait()


def kernel(inputs, Wg, bg, W1, b1, W2, b2):
    rout, hist = pl.pallas_call(
        _gate_kernel,
        out_shape=(
            jax.ShapeDtypeStruct((N_TOKENS, N_EXPERTS), jnp.float32),
            jax.ShapeDtypeStruct((NW, 16), jnp.int32),
        ),
    )(inputs, Wg, bg[None, :])
    rout_flat = rout.reshape(N_TOKENS * N_EXPERTS)
    hist_flat = hist.reshape(NW * 16)

    mesh = plsc.VectorSubcoreMesh(core_axis_name="c", subcore_axis_name="s")

    xg, meta, pos = pl.kernel(
        _sc_dispatch_body,
        out_type=(
            jax.ShapeDtypeStruct((PADTOT, D_MODEL), jnp.float32),
            jax.ShapeDtypeStruct((META_LEN,), jnp.int32),
            jax.ShapeDtypeStruct((2 * N_TOKENS,), jnp.int32),
        ),
        mesh=mesh,
        compiler_params=pltpu.CompilerParams(needs_layout_passes=False),
        scratch_types=[
            pltpu.VMEM((TPW * N_EXPERTS,), jnp.float32),
            pltpu.VMEM((NW * 16,), jnp.int32),
            pltpu.VMEM((TPW,), jnp.int32),
            pltpu.VMEM((TPW,), jnp.int32),
            pltpu.VMEM((TPW, D_MODEL), jnp.float32),
            pltpu.VMEM((META_LEN,), jnp.int32),
            pltpu.SemaphoreType.DMA,
            pltpu.SemaphoreType.DMA,
            pltpu.SemaphoreType.DMA,
        ],
    )(rout_flat, hist_flat, inputs)

    buf = pl.pallas_call(
        _mm_kernel,
        grid_spec=pltpu.PrefetchScalarGridSpec(
            num_scalar_prefetch=1,
            grid=(NBLK,),
            in_specs=[
                pl.BlockSpec((B_ROWS, D_MODEL),
                             lambda b, meta: (jnp.minimum(b, meta[NBLK] - 1), 0)),
                pl.BlockSpec((1, D_MODEL, D_FF), lambda b, meta: (meta[b], 0, 0)),
                pl.BlockSpec((1, 1, D_FF), lambda b, meta: (meta[b], 0, 0)),
                pl.BlockSpec((1, D_FF, D_MODEL), lambda b, meta: (meta[b], 0, 0)),
                pl.BlockSpec((1, 1, D_MODEL), lambda b, meta: (meta[b], 0, 0)),
            ],
            out_specs=pl.BlockSpec(
                (B_ROWS, D_MODEL),
                lambda b, meta: (jnp.minimum(b, meta[NBLK] - 1), 0)),
        ),
        out_shape=jax.ShapeDtypeStruct((PADTOT, D_MODEL), jnp.float32),
    )(meta, xg, W1, b1[:, None, :], W2, b2[:, None, :])

    out = pl.kernel(
        _sc_combine_body,
        out_type=jax.ShapeDtypeStruct((N_TOKENS, D_MODEL), jnp.float32),
        mesh=mesh,
        compiler_params=pltpu.CompilerParams(needs_layout_passes=False),
        scratch_types=[
            pltpu.VMEM((TPW * N_EXPERTS,), jnp.float32),
            pltpu.VMEM((TPW,), jnp.int32),
            pltpu.VMEM((TPW,), jnp.int32),
            pltpu.VMEM((QB, D_MODEL), jnp.float32),
            pltpu.VMEM((QB, D_MODEL), jnp.float32),
            pltpu.VMEM((QB, D_MODEL), jnp.float32),
            pltpu.VMEM((QB, D_MODEL), jnp.float32),
            pltpu.VMEM((QB, D_MODEL), jnp.float32),
            pltpu.VMEM((QB, D_MODEL), jnp.float32),
            pltpu.SemaphoreType.DMA,
            pltpu.SemaphoreType.DMA,
            pltpu.SemaphoreType.DMA,
            pltpu.SemaphoreType.DMA,
            pltpu.SemaphoreType.DMA,
            pltpu.SemaphoreType.DMA,
        ],
    )(buf, pos, rout_flat)
    return out
